# baseline (speedup 1.0000x reference)
import functools

import jax
import jax.numpy as jnp
from jax import lax
from jax.experimental import pallas as pl
from jax.experimental.pallas import tpu as pltpu
from jax.experimental.pallas import tpu_sc as plsc

NUM_EXPERTS = 64
TOP_K = 8
NUM_GROUPS = 8
EXPERTS_PER_GROUP = 8
NUM_LIMITED_GROUPS = 4
ROUTE_SCALE = 2.5

_NEG = float("-inf")

_LANES = 16
_NW = 32


def _scores_body(x_ref, wt_ref, st_ref):
    logits = jnp.dot(x_ref[...], wt_ref[...], preferred_element_type=jnp.float32)
    st_ref[0] = 1.0 / (1.0 + jnp.exp(-logits.T))


def _tc_scores(x, wt):
    nt, d = x.shape
    B = nt // _NW
    return pl.pallas_call(
        _scores_body,
        grid=(_NW,),
        in_specs=[
            pl.BlockSpec((B, d), lambda i: (i, 0)),
            pl.BlockSpec((d, NUM_EXPERTS), lambda i: (0, 0)),
        ],
        out_specs=pl.BlockSpec((1, NUM_EXPERTS, B), lambda i: (i, 0, 0)),
        out_shape=jax.ShapeDtypeStruct((_NW, NUM_EXPERTS, B), jnp.float32),
    )(x, wt)


def _max_tree(vs):
    vs = list(vs)
    while len(vs) > 1:
        vs = [jnp.maximum(vs[i], vs[i + 1]) for i in range(0, len(vs) - 1, 2)] \
            + ([vs[-1]] if len(vs) % 2 else [])
    return vs[0]


def _make_sc_router(nt):
    tpw = nt // _NW
    n_chunks = tpw // _LANES
    mesh = plsc.VectorSubcoreMesh(core_axis_name="c", subcore_axis_name="s")

    @functools.partial(
        pl.kernel, mesh=mesh,
        out_type=[
            jax.ShapeDtypeStruct((_NW, TOP_K, tpw), jnp.float32),
            jax.ShapeDtypeStruct((_NW, TOP_K, tpw), jnp.int32),
            jax.ShapeDtypeStruct((_NW, NUM_EXPERTS, _LANES), jnp.float32),
        ],
        scratch_types=[
            pltpu.VMEM((NUM_EXPERTS, tpw), jnp.float32),
            pltpu.VMEM((NUM_EXPERTS, _LANES), jnp.float32),
            pltpu.VMEM((NUM_EXPERTS, _LANES), jnp.float32),
            pltpu.VMEM((NUM_EXPERTS, _LANES), jnp.float32),
            pltpu.VMEM((NUM_GROUPS, _LANES), jnp.float32),
            pltpu.VMEM((TOP_K, tpw), jnp.float32),
            pltpu.VMEM((TOP_K, tpw), jnp.int32),
            pltpu.VMEM((NUM_EXPERTS, _LANES), jnp.float32),
        ],
        compiler_params=pltpu.CompilerParams(needs_layout_passes=False),
    )
    def sc_router(st_hbm, bias_hbm, ts_hbm, idx_hbm, hist_hbm,
                  sc_v, bias_v, biased_v, sfc_v, gm_v, ts_v, idx_v, hist_v):
        n_cores = 2
        wid = lax.axis_index("s") * n_cores + lax.axis_index("c")
        pltpu.sync_copy(st_hbm.at[wid], sc_v)
        pltpu.sync_copy(bias_hbm, bias_v)
        lane = lax.broadcasted_iota(jnp.int32, (_LANES,), 0)
        zeros = jnp.zeros((_LANES,), jnp.float32)
        for e in range(NUM_EXPERTS):
            hist_v[e, :] = zeros

        def chunk(c, carry):
            col = c * _LANES
            gsum = []
            m1 = m2 = None
            for e in range(NUM_EXPERTS):
                v = sc_v[e, pl.ds(col, _LANES)] + bias_v[e, :]
                biased_v[e, :] = v
                if e % EXPERTS_PER_GROUP == 0:
                    m1, m2 = v, jnp.full((_LANES,), _NEG, jnp.float32)
                else:
                    m2 = jnp.maximum(m2, jnp.minimum(m1, v))
                    m1 = jnp.maximum(m1, v)
                if e % EXPERTS_PER_GROUP == EXPERTS_PER_GROUP - 1:
                    gsum.append(m1 + m2)
            selg = [jnp.zeros((_LANES,), jnp.float32) for _ in range(NUM_GROUPS)]
            work = list(gsum)
            for _ in range(NUM_LIMITED_GROUPS):
                m = _max_tree(work)
                gi = jnp.full((_LANES,), NUM_GROUPS, jnp.int32)
                for g in range(NUM_GROUPS):
                    gi = jnp.minimum(
                        gi, jnp.where(work[g] == m, g, NUM_GROUPS))
                for g in range(NUM_GROUPS):
                    ch = gi == g
                    selg[g] = jnp.maximum(selg[g], ch.astype(jnp.float32))
                    work[g] = jnp.where(ch, _NEG, work[g])
            gmax = None
            for e in range(NUM_EXPERTS):
                s = jnp.where(selg[e // EXPERTS_PER_GROUP] > 0.0,
                              biased_v[e, :], _NEG)
                sfc_v[e, :] = s
                gmax = s if e % EXPERTS_PER_GROUP == 0 else jnp.maximum(gmax, s)
                if e % EXPERTS_PER_GROUP == EXPERTS_PER_GROUP - 1:
                    gm_v[e // EXPERTS_PER_GROUP, :] = gmax
            sck = []
            for k in range(TOP_K):
                gmv = [gm_v[g, :] for g in range(NUM_GROUPS)]
                m = _max_tree(gmv)
                gstar = jnp.full((_LANES,), NUM_GROUPS, jnp.int32)
                for g in range(NUM_GROUPS):
                    gstar = jnp.minimum(
                        gstar, jnp.where(gmv[g] == m, g, NUM_GROUPS))
                row = gstar * EXPERTS_PER_GROUP
                best = jnp.full((_LANES,), _NEG, jnp.float32)
                estar = jnp.full((_LANES,), NUM_EXPERTS, jnp.int32)
                for j in range(EXPERTS_PER_GROUP):
                    r = row + j
                    vj = plsc.load_gather(sfc_v, [r, lane])
                    gt = vj > best
                    estar = jnp.where(gt, r, estar)
                    best = jnp.where(gt, vj, best)
                sck.append(plsc.load_gather(sc_v, [estar, col + lane]))
                plsc.store_scatter(sfc_v, [estar, lane],
                                   jnp.full((_LANES,), _NEG, jnp.float32))
                nm = jnp.full((_LANES,), _NEG, jnp.float32)
                for j in range(EXPERTS_PER_GROUP):
                    nm = jnp.maximum(
                        nm, plsc.load_gather(sfc_v, [row + j, lane]))
                plsc.store_scatter(gm_v, [gstar, lane], nm)
                plsc.addupdate_scatter(hist_v, [estar, lane],
                                       jnp.ones((_LANES,), jnp.float32))
                idx_v[k, pl.ds(col, _LANES)] = estar
            denom = sck[0]
            for k in range(1, TOP_K):
                denom = denom + sck[k]
            inv = ROUTE_SCALE / (denom + 1e-20)
            for k in range(TOP_K):
                ts_v[k, pl.ds(col, _LANES)] = sck[k] * inv
            return carry

        lax.fori_loop(0, n_chunks, chunk, 0)
        pltpu.sync_copy(ts_v, ts_hbm.at[wid])
        pltpu.sync_copy(idx_v, idx_hbm.at[wid])
        pltpu.sync_copy(hist_v, hist_hbm.at[wid])

    return sc_router


def kernel(x, expert_bias, gate_weight):
    nt, _ = x.shape
    wt = gate_weight.T
    st = _tc_scores(x, wt)
    bias_b = jnp.broadcast_to(
        expert_bias.reshape(NUM_EXPERTS, 1), (NUM_EXPERTS, _LANES))
    ts_p, idx_p, hist_p = _make_sc_router(nt)(st, bias_b)
    ts = jnp.transpose(ts_p, (0, 2, 1)).reshape(nt, TOP_K)
    idx = jnp.transpose(idx_p, (0, 2, 1)).reshape(nt, TOP_K)
    counts = jnp.sum(hist_p, axis=(0, 2))
    return ts, idx, counts


# speedup vs baseline: 1.0286x; 1.0286x over previous
import functools

import jax
import jax.numpy as jnp
from jax import lax
from jax.experimental import pallas as pl
from jax.experimental.pallas import tpu as pltpu
from jax.experimental.pallas import tpu_sc as plsc

NUM_EXPERTS = 64
TOP_K = 8
NUM_GROUPS = 8
EXPERTS_PER_GROUP = 8
NUM_LIMITED_GROUPS = 4
ROUTE_SCALE = 2.5

_NEG = float("-inf")

_LANES = 16
_NW = 32
_TPW = 256
_TCB = 1024
_CHUNK = _NW * _TPW


def _scores_body(x_ref, wt_ref, st_ref):
    logits = jnp.dot(x_ref[...], wt_ref[...], preferred_element_type=jnp.float32)
    lt = 1.0 / (1.0 + jnp.exp(-logits.T))
    for j in range(_TCB // _TPW):
        st_ref[j] = lt[:, j * _TPW:(j + 1) * _TPW]


def _tc_scores_chunk(x, wt, c):
    _, d = x.shape
    nb = _TCB // _TPW
    return pl.pallas_call(
        _scores_body,
        grid=(_CHUNK // _TCB,),
        in_specs=[
            pl.BlockSpec((_TCB, d), lambda i: (c * (_CHUNK // _TCB) + i, 0)),
            pl.BlockSpec((d, NUM_EXPERTS), lambda i: (0, 0)),
        ],
        out_specs=pl.BlockSpec((nb, NUM_EXPERTS, _TPW), lambda i: (i, 0, 0)),
        out_shape=jax.ShapeDtypeStruct(
            (_CHUNK // _TPW, NUM_EXPERTS, _TPW), jnp.float32),
    )(x, wt)


def _max_tree(vs):
    vs = list(vs)
    while len(vs) > 1:
        vs = [jnp.maximum(vs[i], vs[i + 1]) for i in range(0, len(vs) - 1, 2)] \
            + ([vs[-1]] if len(vs) % 2 else [])
    return vs[0]


def _make_sc_router():
    tpw = _TPW
    n_chunks = tpw // _LANES
    mesh = plsc.VectorSubcoreMesh(core_axis_name="c", subcore_axis_name="s")

    @functools.partial(
        pl.kernel, mesh=mesh,
        out_type=[
            jax.ShapeDtypeStruct((_NW, TOP_K, tpw), jnp.float32),
            jax.ShapeDtypeStruct((_NW, TOP_K, tpw), jnp.int32),
            jax.ShapeDtypeStruct((_NW, NUM_EXPERTS, _LANES), jnp.float32),
        ],
        scratch_types=[
            pltpu.VMEM((NUM_EXPERTS, tpw), jnp.float32),
            pltpu.VMEM((NUM_EXPERTS, _LANES), jnp.float32),
            pltpu.VMEM((NUM_EXPERTS, _LANES), jnp.float32),
            pltpu.VMEM((NUM_EXPERTS, _LANES), jnp.float32),
            pltpu.VMEM((NUM_GROUPS, _LANES), jnp.float32),
            pltpu.VMEM((TOP_K, tpw), jnp.float32),
            pltpu.VMEM((TOP_K, tpw), jnp.int32),
            pltpu.VMEM((NUM_EXPERTS, _LANES), jnp.float32),
        ],
        compiler_params=pltpu.CompilerParams(needs_layout_passes=False),
    )
    def sc_router(st_hbm, bias_hbm, ts_hbm, idx_hbm, hist_hbm,
                  sc_v, bias_v, biased_v, sfc_v, gm_v, ts_v, idx_v, hist_v):
        n_cores = 2
        wid = lax.axis_index("s") * n_cores + lax.axis_index("c")
        pltpu.sync_copy(st_hbm.at[wid], sc_v)
        pltpu.sync_copy(bias_hbm, bias_v)
        lane = lax.broadcasted_iota(jnp.int32, (_LANES,), 0)
        zeros = jnp.zeros((_LANES,), jnp.float32)
        for e in range(NUM_EXPERTS):
            hist_v[e, :] = zeros

        def chunk(c, carry):
            col = c * _LANES
            gsum = []
            m1 = m2 = None
            for e in range(NUM_EXPERTS):
                v = sc_v[e, pl.ds(col, _LANES)] + bias_v[e, :]
                biased_v[e, :] = v
                if e % EXPERTS_PER_GROUP == 0:
                    m1, m2 = v, jnp.full((_LANES,), _NEG, jnp.float32)
                else:
                    m2 = jnp.maximum(m2, jnp.minimum(m1, v))
                    m1 = jnp.maximum(m1, v)
                if e % EXPERTS_PER_GROUP == EXPERTS_PER_GROUP - 1:
                    gsum.append(m1 + m2)
            selg = [jnp.zeros((_LANES,), jnp.float32) for _ in range(NUM_GROUPS)]
            work = list(gsum)
            for _ in range(NUM_LIMITED_GROUPS):
                m = _max_tree(work)
                gi = jnp.full((_LANES,), NUM_GROUPS, jnp.int32)
                for g in range(NUM_GROUPS):
                    gi = jnp.minimum(
                        gi, jnp.where(work[g] == m, g, NUM_GROUPS))
                for g in range(NUM_GROUPS):
                    ch = gi == g
                    selg[g] = jnp.maximum(selg[g], ch.astype(jnp.float32))
                    work[g] = jnp.where(ch, _NEG, work[g])
            gmax = None
            for e in range(NUM_EXPERTS):
                s = jnp.where(selg[e // EXPERTS_PER_GROUP] > 0.0,
                              biased_v[e, :], _NEG)
                sfc_v[e, :] = s
                gmax = s if e % EXPERTS_PER_GROUP == 0 else jnp.maximum(gmax, s)
                if e % EXPERTS_PER_GROUP == EXPERTS_PER_GROUP - 1:
                    gm_v[e // EXPERTS_PER_GROUP, :] = gmax
            sck = []
            for k in range(TOP_K):
                gmv = [gm_v[g, :] for g in range(NUM_GROUPS)]
                m = _max_tree(gmv)
                gstar = jnp.full((_LANES,), NUM_GROUPS, jnp.int32)
                for g in range(NUM_GROUPS):
                    gstar = jnp.minimum(
                        gstar, jnp.where(gmv[g] == m, g, NUM_GROUPS))
                row = gstar * EXPERTS_PER_GROUP
                best = jnp.full((_LANES,), _NEG, jnp.float32)
                estar = jnp.full((_LANES,), NUM_EXPERTS, jnp.int32)
                for j in range(EXPERTS_PER_GROUP):
                    r = row + j
                    vj = plsc.load_gather(sfc_v, [r, lane])
                    gt = vj > best
                    estar = jnp.where(gt, r, estar)
                    best = jnp.where(gt, vj, best)
                sck.append(plsc.load_gather(sc_v, [estar, col + lane]))
                plsc.store_scatter(sfc_v, [estar, lane],
                                   jnp.full((_LANES,), _NEG, jnp.float32))
                nm = jnp.full((_LANES,), _NEG, jnp.float32)
                for j in range(EXPERTS_PER_GROUP):
                    nm = jnp.maximum(
                        nm, plsc.load_gather(sfc_v, [row + j, lane]))
                plsc.store_scatter(gm_v, [gstar, lane], nm)
                plsc.addupdate_scatter(hist_v, [estar, lane],
                                       jnp.ones((_LANES,), jnp.float32))
                idx_v[k, pl.ds(col, _LANES)] = estar
            denom = sck[0]
            for k in range(1, TOP_K):
                denom = denom + sck[k]
            inv = ROUTE_SCALE / (denom + 1e-20)
            for k in range(TOP_K):
                ts_v[k, pl.ds(col, _LANES)] = sck[k] * inv
            return carry

        lax.fori_loop(0, n_chunks, chunk, 0)
        pltpu.sync_copy(ts_v, ts_hbm.at[wid])
        pltpu.sync_copy(idx_v, idx_hbm.at[wid])
        pltpu.sync_copy(hist_v, hist_hbm.at[wid])

    return sc_router


def kernel(x, expert_bias, gate_weight):
    nt, _ = x.shape
    wt = gate_weight.T
    bias_b = jnp.broadcast_to(
        expert_bias.reshape(NUM_EXPERTS, 1), (NUM_EXPERTS, _LANES))
    router = _make_sc_router()
    ts_parts, idx_parts, hist_parts = [], [], []
    for c in range(nt // _CHUNK):
        st = _tc_scores_chunk(x, wt, c)
        ts_p, idx_p, hist_p = router(st, bias_b)
        ts_parts.append(ts_p)
        idx_parts.append(idx_p)
        hist_parts.append(hist_p)
    ts = jnp.concatenate(ts_parts, axis=0)
    idx = jnp.concatenate(idx_parts, axis=0)
    ts = jnp.transpose(ts, (0, 2, 1)).reshape(nt, TOP_K)
    idx = jnp.transpose(idx, (0, 2, 1)).reshape(nt, TOP_K)
    counts = sum(jnp.sum(h, axis=(0, 2)) for h in hist_parts)
    return ts, idx, counts


# speedup vs baseline: 1.0315x; 1.0028x over previous
import functools

import jax
import jax.numpy as jnp
from jax import lax
from jax.experimental import pallas as pl
from jax.experimental.pallas import tpu as pltpu
from jax.experimental.pallas import tpu_sc as plsc

NUM_EXPERTS = 64
TOP_K = 8
NUM_GROUPS = 8
EXPERTS_PER_GROUP = 8
NUM_LIMITED_GROUPS = 4
ROUTE_SCALE = 2.5

_NEG = float("-inf")

_LANES = 16
_NW = 32
_TPW = 256
_TCB = 1024
_CHUNK = _NW * _TPW


def _scores_body(x_ref, wt_ref, st_ref):
    logits = jnp.dot(x_ref[...], wt_ref[...], preferred_element_type=jnp.float32)
    lt = 1.0 / (1.0 + jnp.exp(-logits.T))
    for j in range(_TCB // _TPW):
        st_ref[j] = lt[:, j * _TPW:(j + 1) * _TPW]


def _tc_scores_chunk(x, wt, c):
    _, d = x.shape
    nb = _TCB // _TPW
    return pl.pallas_call(
        _scores_body,
        grid=(_CHUNK // _TCB,),
        in_specs=[
            pl.BlockSpec((_TCB, d), lambda i: (c * (_CHUNK // _TCB) + i, 0)),
            pl.BlockSpec((d, NUM_EXPERTS), lambda i: (0, 0)),
        ],
        out_specs=pl.BlockSpec((nb, NUM_EXPERTS, _TPW), lambda i: (i, 0, 0)),
        out_shape=jax.ShapeDtypeStruct(
            (_CHUNK // _TPW, NUM_EXPERTS, _TPW), jnp.float32),
    )(x, wt)


def _max_tree(vs):
    vs = list(vs)
    while len(vs) > 1:
        vs = [jnp.maximum(vs[i], vs[i + 1]) for i in range(0, len(vs) - 1, 2)] \
            + ([vs[-1]] if len(vs) % 2 else [])
    return vs[0]


def _make_sc_router():
    tpw = _TPW
    n_chunks = tpw // _LANES
    mesh = plsc.VectorSubcoreMesh(core_axis_name="c", subcore_axis_name="s")

    @functools.partial(
        pl.kernel, mesh=mesh,
        out_type=[
            jax.ShapeDtypeStruct((_NW, TOP_K, tpw), jnp.float32),
            jax.ShapeDtypeStruct((_NW, TOP_K, tpw), jnp.int32),
            jax.ShapeDtypeStruct((_NW, NUM_EXPERTS, _LANES), jnp.float32),
        ],
        scratch_types=[
            pltpu.VMEM((NUM_EXPERTS, tpw), jnp.float32),
            pltpu.VMEM((NUM_EXPERTS, _LANES), jnp.float32),
            pltpu.VMEM((NUM_EXPERTS, _LANES), jnp.float32),
            pltpu.VMEM((NUM_EXPERTS, _LANES), jnp.float32),
            pltpu.VMEM((NUM_GROUPS, _LANES), jnp.float32),
            pltpu.VMEM((TOP_K, tpw), jnp.float32),
            pltpu.VMEM((TOP_K, tpw), jnp.int32),
            pltpu.VMEM((NUM_EXPERTS, _LANES), jnp.float32),
        ],
        compiler_params=pltpu.CompilerParams(needs_layout_passes=False),
    )
    def sc_router(st_hbm, bias_hbm, ts_hbm, idx_hbm, hist_hbm,
                  sc_v, bias_v, biased_v, sfc_v, gm_v, ts_v, idx_v, hist_v):
        n_cores = 2
        wid = lax.axis_index("s") * n_cores + lax.axis_index("c")
        pltpu.sync_copy(st_hbm.at[wid], sc_v)
        pltpu.sync_copy(bias_hbm, bias_v)
        lane = lax.broadcasted_iota(jnp.int32, (_LANES,), 0)
        zeros = jnp.zeros((_LANES,), jnp.float32)
        for e in range(NUM_EXPERTS):
            hist_v[e, :] = zeros

        def chunk(c, carry):
            col = c * _LANES
            gsum = []
            m1 = m2 = None
            for e in range(NUM_EXPERTS):
                v = sc_v[e, pl.ds(col, _LANES)] + bias_v[e, :]
                biased_v[e, :] = v
                if e % EXPERTS_PER_GROUP == 0:
                    m1, m2 = v, jnp.full((_LANES,), _NEG, jnp.float32)
                else:
                    m2 = jnp.maximum(m2, jnp.minimum(m1, v))
                    m1 = jnp.maximum(m1, v)
                if e % EXPERTS_PER_GROUP == EXPERTS_PER_GROUP - 1:
                    gsum.append(m1 + m2)
            selg = [jnp.zeros((_LANES,), jnp.float32) for _ in range(NUM_GROUPS)]
            work = list(gsum)
            for _ in range(NUM_LIMITED_GROUPS):
                m = _max_tree(work)
                gi = jnp.full((_LANES,), NUM_GROUPS, jnp.int32)
                for g in range(NUM_GROUPS):
                    gi = jnp.minimum(
                        gi, jnp.where(work[g] == m, g, NUM_GROUPS))
                for g in range(NUM_GROUPS):
                    ch = gi == g
                    selg[g] = jnp.maximum(selg[g], ch.astype(jnp.float32))
                    work[g] = jnp.where(ch, _NEG, work[g])
            gmax = None
            for e in range(NUM_EXPERTS):
                s = jnp.where(selg[e // EXPERTS_PER_GROUP] > 0.0,
                              biased_v[e, :], _NEG)
                sfc_v[e, :] = s
                gmax = s if e % EXPERTS_PER_GROUP == 0 else jnp.maximum(gmax, s)
                if e % EXPERTS_PER_GROUP == EXPERTS_PER_GROUP - 1:
                    gm_v[e // EXPERTS_PER_GROUP, :] = gmax
            sck = []
            for k in range(TOP_K):
                gmv = [gm_v[g, :] for g in range(NUM_GROUPS)]
                m = _max_tree(gmv)
                gstar = jnp.full((_LANES,), NUM_GROUPS, jnp.int32)
                for g in range(NUM_GROUPS):
                    gstar = jnp.minimum(
                        gstar, jnp.where(gmv[g] == m, g, NUM_GROUPS))
                row = gstar * EXPERTS_PER_GROUP
                best = jnp.full((_LANES,), _NEG, jnp.float32)
                best2 = jnp.full((_LANES,), _NEG, jnp.float32)
                estar = jnp.full((_LANES,), NUM_EXPERTS, jnp.int32)
                for j in range(EXPERTS_PER_GROUP):
                    r = row + j
                    vj = plsc.load_gather(sfc_v, [r, lane])
                    gt = vj > best
                    estar = jnp.where(gt, r, estar)
                    best2 = jnp.maximum(best2, jnp.minimum(best, vj))
                    best = jnp.where(gt, vj, best)
                sck.append(plsc.load_gather(sc_v, [estar, col + lane]))
                plsc.store_scatter(sfc_v, [estar, lane],
                                   jnp.full((_LANES,), _NEG, jnp.float32))
                plsc.store_scatter(gm_v, [gstar, lane], best2)
                plsc.addupdate_scatter(hist_v, [estar, lane],
                                       jnp.ones((_LANES,), jnp.float32))
                idx_v[k, pl.ds(col, _LANES)] = estar
            denom = sck[0]
            for k in range(1, TOP_K):
                denom = denom + sck[k]
            inv = ROUTE_SCALE / (denom + 1e-20)
            for k in range(TOP_K):
                ts_v[k, pl.ds(col, _LANES)] = sck[k] * inv
            return carry

        lax.fori_loop(0, n_chunks, chunk, 0)
        pltpu.sync_copy(ts_v, ts_hbm.at[wid])
        pltpu.sync_copy(idx_v, idx_hbm.at[wid])
        pltpu.sync_copy(hist_v, hist_hbm.at[wid])

    return sc_router


def kernel(x, expert_bias, gate_weight):
    nt, _ = x.shape
    wt = gate_weight.T
    bias_b = jnp.broadcast_to(
        expert_bias.reshape(NUM_EXPERTS, 1), (NUM_EXPERTS, _LANES))
    router = _make_sc_router()
    ts_parts, idx_parts, hist_parts = [], [], []
    for c in range(nt // _CHUNK):
        st = _tc_scores_chunk(x, wt, c)
        ts_p, idx_p, hist_p = router(st, bias_b)
        ts_parts.append(ts_p)
        idx_parts.append(idx_p)
        hist_parts.append(hist_p)
    ts = jnp.concatenate(ts_parts, axis=0)
    idx = jnp.concatenate(idx_parts, axis=0)
    ts = jnp.transpose(ts, (0, 2, 1)).reshape(nt, TOP_K)
    idx = jnp.transpose(idx, (0, 2, 1)).reshape(nt, TOP_K)
    counts = sum(jnp.sum(h, axis=(0, 2)) for h in hist_parts)
    return ts, idx, counts
